# Initial kernel scaffold; baseline (speedup 1.0000x reference)
import jax
import jax.numpy as jnp
from jax.experimental import pallas as pl


def kernel(x):
    raise NotImplementedError("write your pallas kernel here")


# speedup vs baseline: 98.8814x; 98.8814x over previous
import jax
import jax.numpy as jnp
from jax import lax
from jax.experimental import pallas as pl
from jax.experimental.pallas import tpu as pltpu

_K = 8
_UNROLL = 8


def _insert(lst, v):
    out = []
    for t in lst:
        hi = jnp.maximum(t, v)
        v = jnp.minimum(t, v)
        out.append(hi)
    return out


def _merge_top8(l, r):
    return [jnp.maximum(l[j], r[7 - j]) for j in range(8)]


def _bitonic_sort8(m):
    for d in (4, 2, 1):
        nm = list(m)
        for j in range(8):
            if (j % (2 * d)) < d:
                nm[j] = jnp.maximum(m[j], m[j + d])
                nm[j + d] = jnp.minimum(m[j], m[j + d])
        m = nm
    return m


def _body(x_ref, o_ref):
    nvreg = x_ref.shape[1]
    c = x_ref.shape[3]
    init = jnp.full((8, c), -jnp.inf, jnp.float32)
    ta = tuple([init] * 8)
    tb = tuple([init] * 8)

    def step(i, carry):
        ta, tb = list(carry[0]), list(carry[1])
        for u in range(_UNROLL // 2):
            va = x_ref[0, i * _UNROLL + 2 * u]
            vb = x_ref[0, i * _UNROLL + 2 * u + 1]
            ta = _insert(ta, va)
            tb = _insert(tb, vb)
        return (tuple(ta), tuple(tb))

    ta, tb = lax.fori_loop(0, nvreg // _UNROLL, step, (ta, tb))

    t = _bitonic_sort8(_merge_top8(list(ta), list(tb)))
    for d in (4, 2):
        r = [pltpu.roll(a, d, 0) for a in t]
        t = _bitonic_sort8(_merge_top8(t, r))
    r = [pltpu.roll(a, 1, 0) for a in t]
    m = _merge_top8(t, r)
    s = m[0]
    for j in range(1, 8):
        s = s + m[j]
    s = s * jnp.float32(1.0 / _K)
    o_ref[0, :, :] = s[0:1, :]


def kernel(x):
    b, s, c = x.shape
    xr = x.reshape(b, s // 8, 8, c)
    out = pl.pallas_call(
        _body,
        grid=(b,),
        in_specs=[pl.BlockSpec((1, s // 8, 8, c), lambda i: (i, 0, 0, 0))],
        out_specs=pl.BlockSpec((1, 1, c), lambda i: (i, 0, 0)),
        out_shape=jax.ShapeDtypeStruct((b, 1, c), jnp.float32),
    )(xr)
    return out.reshape(b, c)


# speedup vs baseline: 133.6142x; 1.3513x over previous
import jax
import jax.numpy as jnp
from jax import lax
from jax.experimental import pallas as pl
from jax.experimental.pallas import tpu as pltpu

_K = 8
_UNROLL = 8


_SORT8_NET = (
    (0, 2), (1, 3), (4, 6), (5, 7),
    (0, 4), (1, 5), (2, 6), (3, 7),
    (0, 1), (2, 3), (4, 5), (6, 7),
    (2, 4), (3, 5),
    (1, 4), (3, 6),
    (1, 2), (3, 4), (5, 6),
)


def _sort8_desc(vs):
    vs = list(vs)
    for i, j in _SORT8_NET:
        hi = jnp.maximum(vs[i], vs[j])
        lo = jnp.minimum(vs[i], vs[j])
        vs[i], vs[j] = hi, lo
    return vs


def _merge_top8(l, r):
    return [jnp.maximum(l[j], r[7 - j]) for j in range(8)]


def _bitonic_sort8(m):
    for d in (4, 2, 1):
        nm = list(m)
        for j in range(8):
            if (j % (2 * d)) < d:
                nm[j] = jnp.maximum(m[j], m[j + d])
                nm[j + d] = jnp.minimum(m[j], m[j + d])
        m = nm
    return m


def _body(x_ref, o_ref):
    nvreg = x_ref.shape[1]
    c = x_ref.shape[3]
    init = jnp.full((8, c), -jnp.inf, jnp.float32)

    def step(i, t):
        vs = [x_ref[0, i * _UNROLL + u] for u in range(_UNROLL)]
        s = _sort8_desc(vs)
        return tuple(_bitonic_sort8(_merge_top8(list(t), s)))

    t = lax.fori_loop(0, nvreg // _UNROLL, step, tuple([init] * 8))
    t = list(t)
    for d in (4, 2):
        r = [pltpu.roll(a, d, 0) for a in t]
        t = _bitonic_sort8(_merge_top8(t, r))
    r = [pltpu.roll(a, 1, 0) for a in t]
    m = _merge_top8(t, r)
    s = m[0]
    for j in range(1, 8):
        s = s + m[j]
    s = s * jnp.float32(1.0 / _K)
    o_ref[0, :, :] = s[0:1, :]


def kernel(x):
    b, s, c = x.shape
    xr = x.reshape(b, s // 8, 8, c)
    out = pl.pallas_call(
        _body,
        grid=(b,),
        in_specs=[pl.BlockSpec((1, s // 8, 8, c), lambda i: (i, 0, 0, 0))],
        out_specs=pl.BlockSpec((1, 1, c), lambda i: (i, 0, 0)),
        out_shape=jax.ShapeDtypeStruct((b, 1, c), jnp.float32),
    )(xr)
    return out.reshape(b, c)


# speedup vs baseline: 139.9896x; 1.0477x over previous
import jax
import jax.numpy as jnp
from jax import lax
from jax.experimental import pallas as pl
from jax.experimental.pallas import tpu as pltpu

_K = 8
_UNROLL = 8


_SORT8_NET = (
    (0, 2), (1, 3), (4, 6), (5, 7),
    (0, 4), (1, 5), (2, 6), (3, 7),
    (0, 1), (2, 3), (4, 5), (6, 7),
    (2, 4), (3, 5),
    (1, 4), (3, 6),
    (1, 2), (3, 4), (5, 6),
)


def _sort8_desc(vs):
    vs = list(vs)
    for i, j in _SORT8_NET:
        hi = jnp.maximum(vs[i], vs[j])
        lo = jnp.minimum(vs[i], vs[j])
        vs[i], vs[j] = hi, lo
    return vs


def _merge_top8(l, r):
    return [jnp.maximum(l[j], r[7 - j]) for j in range(8)]


def _bitonic_sort8(m):
    for d in (4, 2, 1):
        nm = list(m)
        for j in range(8):
            if (j % (2 * d)) < d:
                nm[j] = jnp.maximum(m[j], m[j + d])
                nm[j + d] = jnp.minimum(m[j], m[j + d])
        m = nm
    return m


_NB = 4


def _body(x_ref, o_ref):
    nvreg = x_ref.shape[1]
    c = x_ref.shape[3]
    init = jnp.full((8, c), -jnp.inf, jnp.float32)

    for bb in range(_NB):
        def step(i, t):
            vs = [x_ref[bb, i * _UNROLL + u] for u in range(_UNROLL)]
            s = _sort8_desc(vs)
            return tuple(_bitonic_sort8(_merge_top8(list(t), s)))

        t = lax.fori_loop(0, nvreg // _UNROLL, step, tuple([init] * 8))
        t = list(t)
        for d in (4, 2):
            r = [pltpu.roll(a, d, 0) for a in t]
            t = _bitonic_sort8(_merge_top8(t, r))
        r = [pltpu.roll(a, 1, 0) for a in t]
        m = _merge_top8(t, r)
        s = m[0]
        for j in range(1, 8):
            s = s + m[j]
        s = s * jnp.float32(1.0 / _K)
        o_ref[bb, :, :] = s[0:1, :]


def kernel(x):
    b, s, c = x.shape
    xr = x.reshape(b, s // 8, 8, c)
    out = pl.pallas_call(
        _body,
        grid=(b // _NB,),
        in_specs=[pl.BlockSpec((_NB, s // 8, 8, c), lambda i: (i, 0, 0, 0))],
        out_specs=pl.BlockSpec((_NB, 1, c), lambda i: (i, 0, 0)),
        out_shape=jax.ShapeDtypeStruct((b, 1, c), jnp.float32),
    )(xr)
    return out.reshape(b, c)


# speedup vs baseline: 174.4407x; 1.2461x over previous
import jax
import jax.numpy as jnp
from jax import lax
from jax.experimental import pallas as pl
from jax.experimental.pallas import tpu as pltpu

_K = 8
_UNROLL = 8


_SORT8_NET = (
    (0, 2), (1, 3), (4, 6), (5, 7),
    (0, 4), (1, 5), (2, 6), (3, 7),
    (0, 1), (2, 3), (4, 5), (6, 7),
    (2, 4), (3, 5),
    (1, 4), (3, 6),
    (1, 2), (3, 4), (5, 6),
)


def _sort8_desc(vs):
    vs = list(vs)
    for i, j in _SORT8_NET:
        hi = jnp.maximum(vs[i], vs[j])
        lo = jnp.minimum(vs[i], vs[j])
        vs[i], vs[j] = hi, lo
    return vs


def _merge_top8(l, r):
    return [jnp.maximum(l[j], r[7 - j]) for j in range(8)]


def _bitonic_sort8(m):
    for d in (4, 2, 1):
        nm = list(m)
        for j in range(8):
            if (j % (2 * d)) < d:
                nm[j] = jnp.maximum(m[j], m[j + d])
                nm[j + d] = jnp.minimum(m[j], m[j + d])
        m = nm
    return m


_NB = 4


def _body(x_ref, o_ref):
    nvreg = x_ref.shape[1]
    c = x_ref.shape[3]
    init = jnp.full((8, c), -jnp.inf, jnp.float32)

    for bb in range(_NB):
        def step(i, carry):
            ta, tb = carry
            va = [x_ref[bb, i * 2 * _UNROLL + u] for u in range(_UNROLL)]
            vb = [x_ref[bb, i * 2 * _UNROLL + _UNROLL + u] for u in range(_UNROLL)]
            sa = _sort8_desc(va)
            sb = _sort8_desc(vb)
            ta = tuple(_bitonic_sort8(_merge_top8(list(ta), sa)))
            tb = tuple(_bitonic_sort8(_merge_top8(list(tb), sb)))
            return (ta, tb)

        t0 = tuple([init] * 8)
        ta, tb = lax.fori_loop(0, nvreg // (2 * _UNROLL), step, (t0, t0))
        t = _bitonic_sort8(_merge_top8(list(ta), list(tb)))
        for d in (4, 2):
            r = [pltpu.roll(a, d, 0) for a in t]
            t = _bitonic_sort8(_merge_top8(t, r))
        r = [pltpu.roll(a, 1, 0) for a in t]
        m = _merge_top8(t, r)
        s = m[0]
        for j in range(1, 8):
            s = s + m[j]
        s = s * jnp.float32(1.0 / _K)
        o_ref[bb, :, :] = s[0:1, :]


def kernel(x):
    b, s, c = x.shape
    xr = x.reshape(b, s // 8, 8, c)
    out = pl.pallas_call(
        _body,
        grid=(b // _NB,),
        in_specs=[pl.BlockSpec((_NB, s // 8, 8, c), lambda i: (i, 0, 0, 0))],
        out_specs=pl.BlockSpec((_NB, 1, c), lambda i: (i, 0, 0)),
        out_shape=jax.ShapeDtypeStruct((b, 1, c), jnp.float32),
    )(xr)
    return out.reshape(b, c)
